# Initial kernel scaffold; baseline (speedup 1.0000x reference)
import jax
import jax.numpy as jnp
from jax.experimental import pallas as pl


def kernel(x, edge_index, edge_attr, pos, strata_data, batch, k, W1, b1, W2, b2, W3, b3, lw1, lb1, lw2, lb2, lw3, lb3):
    raise NotImplementedError("write your pallas kernel here")


# speedup vs baseline: 9.1159x; 9.1159x over previous
import functools

import jax
import jax.numpy as jnp
from jax import lax
from jax.experimental import pallas as pl
from jax.experimental.pallas import tpu as pltpu
from jax.experimental.pallas import tpu_sc as plsc

_N = 10000
_E = 320000
_D = 128
_NC = 2
_NS = 16
_NW = _NC * _NS
_B = 80
_CPW = _E // (_NW * _B)
_RB = 624
_TAIL = _N - _NS * _RB
_ZR = 16
_NZ = _RB // _ZR
_L = 16


def _sc_mesh():
    return plsc.VectorSubcoreMesh(core_axis_name="c", subcore_axis_name="s")


def _zero_acc(z_v, acc_sh, sid, width):
    for r in range(_ZR):
        for c in range(width // _L):
            z_v[r, pl.ds(c * _L, _L)] = jnp.zeros((_L,), jnp.float32)

    def zloop(i, carry):
        pltpu.sync_copy(z_v, acc_sh.at[pl.ds(sid * _RB + i * _ZR, _ZR)])
        return carry

    lax.fori_loop(0, _NZ, zloop, 0)

    @pl.when(sid == 0)
    def _():
        pltpu.sync_copy(z_v, acc_sh.at[pl.ds(_NS * _RB, _TAIL)])


def _dump_acc(acc_sh, out_hbm, cid, sid):
    pltpu.sync_copy(acc_sh.at[pl.ds(sid * _RB, _RB)],
                    out_hbm.at[cid, pl.ds(sid * _RB, _RB)])

    @pl.when(sid == 0)
    def _():
        pltpu.sync_copy(acc_sh.at[pl.ds(_NS * _RB, _TAIL)],
                        out_hbm.at[cid, pl.ds(_NS * _RB, _TAIL)])


@functools.partial(
    pl.kernel,
    mesh=_sc_mesh(),
    out_type=jax.ShapeDtypeStruct((_NC, _N, _D), jnp.float32),
    scratch_types=[
        pltpu.VMEM((_B,), jnp.int32),
        pltpu.VMEM((_B,), jnp.int32),
        pltpu.VMEM((_B,), jnp.float32),
        pltpu.VMEM((2 * _L,), jnp.float32),
        pltpu.VMEM((_N,), jnp.float32),
        pltpu.VMEM((_B, _D), jnp.float32),
        pltpu.VMEM((_ZR, _D), jnp.float32),
        pltpu.VMEM_SHARED((_N, _D), jnp.float32),
        pltpu.SemaphoreType.DMA,
    ],
    compiler_params=pltpu.CompilerParams(needs_layout_passes=False),
)
def _feat_agg(h_hbm, src_hbm, dst_hbm, ew_hbm, a_hbm, out_hbm,
              src_v, dst_v, ew_v, w_v, a_v, rows_v, z_v, acc_sh, sem):
    cid = lax.axis_index("c")
    sid = lax.axis_index("s")
    wid = sid * _NC + cid

    _zero_acc(z_v, acc_sh, sid, _D)
    pltpu.sync_copy(a_hbm, a_v)
    plsc.subcore_barrier()

    base0 = wid * (_E // _NW)

    def chunk(i, carry):
        base = base0 + i * _B
        pltpu.sync_copy(src_hbm.at[pl.ds(base, _B)], src_v)
        pltpu.sync_copy(dst_hbm.at[pl.ds(base, _B)], dst_v)
        pltpu.sync_copy(ew_hbm.at[pl.ds(base, _B)], ew_v)
        pltpu.async_copy(h_hbm.at[src_v], rows_v, sem).wait()
        for g in range(_B // _L):
            s16 = src_v[pl.ds(g * _L, _L)]
            e16 = ew_v[pl.ds(g * _L, _L)]
            w16 = e16 * plsc.load_gather(a_v, [s16])
            w_v[pl.ds(_L, _L)] = w16
            for e in range(_L):
                wb = plsc.load_gather(w_v, [jnp.full((_L,), _L + e, jnp.int32)])
                row = g * _L + e
                for c in range(_D // _L):
                    sl = pl.ds(c * _L, _L)
                    rows_v[row, sl] = rows_v[row, sl] * wb
        pltpu.sync_copy(rows_v, acc_sh.at[dst_v], add=True)
        return carry

    lax.fori_loop(0, _CPW, chunk, 0)
    plsc.subcore_barrier()
    _dump_acc(acc_sh, out_hbm, cid, sid)


@functools.partial(
    pl.kernel,
    mesh=_sc_mesh(),
    out_type=(jax.ShapeDtypeStruct((_E,), jnp.float32),
              jax.ShapeDtypeStruct((_NC, _N, _D), jnp.float32)),
    scratch_types=[
        pltpu.VMEM((_B,), jnp.int32),
        pltpu.VMEM((_B,), jnp.int32),
        pltpu.VMEM((_B,), jnp.float32),
        pltpu.VMEM((_B,), jnp.float32),
        pltpu.VMEM((2 * _L,), jnp.float32),
        pltpu.VMEM((_N,), jnp.float32),
        pltpu.VMEM((_B, _D), jnp.float32),
        pltpu.VMEM((_ZR, _D), jnp.float32),
        pltpu.VMEM_SHARED((_N, _D), jnp.float32),
    ],
    compiler_params=pltpu.CompilerParams(needs_layout_passes=False),
)
def _deg_pass(src_hbm, dst_hbm, ew_hbm, m_hbm, ewout_hbm, deg_hbm,
              src_v, dst_v, ew_v, w_v, stg_v, m_v, rows_v, z_v, acc_sh):
    cid = lax.axis_index("c")
    sid = lax.axis_index("s")
    wid = sid * _NC + cid

    _zero_acc(z_v, acc_sh, sid, _D)
    pltpu.sync_copy(m_hbm, m_v)
    plsc.subcore_barrier()

    base0 = wid * (_E // _NW)

    def chunk(i, carry):
        base = base0 + i * _B
        pltpu.sync_copy(src_hbm.at[pl.ds(base, _B)], src_v)
        pltpu.sync_copy(dst_hbm.at[pl.ds(base, _B)], dst_v)
        pltpu.sync_copy(ew_hbm.at[pl.ds(base, _B)], ew_v)
        for g in range(_B // _L):
            s16 = src_v[pl.ds(g * _L, _L)]
            d16 = dst_v[pl.ds(g * _L, _L)]
            e16 = ew_v[pl.ds(g * _L, _L)]
            w16 = e16 * plsc.load_gather(m_v, [s16]) * plsc.load_gather(m_v, [d16])
            w_v[pl.ds(g * _L, _L)] = w16
            stg_v[pl.ds(_L, _L)] = w16
            for e in range(_L):
                row = g * _L + e
                wb = plsc.load_gather(stg_v, [jnp.full((_L,), _L + e, jnp.int32)])
                for c in range(_D // _L):
                    rows_v[row, pl.ds(c * _L, _L)] = wb
        pltpu.sync_copy(w_v, ewout_hbm.at[pl.ds(base, _B)])
        pltpu.sync_copy(rows_v, acc_sh.at[dst_v], add=True)
        return carry

    lax.fori_loop(0, _CPW, chunk, 0)
    plsc.subcore_barrier()
    _dump_acc(acc_sh, deg_hbm, cid, sid)


def _mm_dinv(h, W, degp):
    n = h.shape[0]

    def body(h_ref, w_ref, degp_ref, hw_ref, dinv_ref):
        hw_ref[...] = jnp.dot(h_ref[...], w_ref[...],
                              preferred_element_type=jnp.float32)
        degn = degp_ref[0, :, 0:1] + degp_ref[1, :, 0:1]
        dinv_ref[...] = lax.rsqrt(degn + 1.0)

    return pl.pallas_call(
        body,
        out_shape=(jax.ShapeDtypeStruct((n, _D), jnp.float32),
                   jax.ShapeDtypeStruct((n, 1), jnp.float32)),
    )(h, W, degp)


def _combine(aggp, hw, dinv, b):
    n = hw.shape[0]

    def body(aggp_ref, hw_ref, dinv_ref, b_ref, out_ref):
        agg = aggp_ref[0] + aggp_ref[1]
        dv = dinv_ref[...]
        out_ref[...] = jax.nn.relu(dv * agg + dv * dv * hw_ref[...] + b_ref[...])

    return pl.pallas_call(
        body,
        out_shape=jax.ShapeDtypeStruct((n, _D), jnp.float32),
    )(aggp, hw, dinv, b)


def _pool(h, aggp, degp, m, kk):
    n = h.shape[0]

    def body(h_ref, aggp_ref, degp_ref, m_ref, mn_ref, hp_ref, xr_ref):
        degn = degp_ref[0, :, 0:1] + degp_ref[1, :, 0:1]
        degs = jnp.where(degn > 0, degn, 1.0)
        neigh = (aggp_ref[0] + aggp_ref[1]) / degs
        hh = h_ref[...]
        sc = jnp.sum(jnp.abs(hh - neigh), axis=1, keepdims=True)
        sbits = jnp.where(m_ref[...] > 0,
                          lax.bitcast_convert_type(sc, jnp.int32),
                          jnp.int32(-1))

        def bis(i, lohi):
            lo, hi = lohi
            mid = lo + (hi - lo + 1) // 2
            cnt = jnp.sum((sbits >= mid).astype(jnp.int32))
            ok = cnt >= kk
            return (jnp.where(ok, mid, lo), jnp.where(ok, hi, mid - 1))

        lo, _ = lax.fori_loop(0, 31, bis,
                              (jnp.int32(0), jnp.int32(0x7F800000)))
        mnew = (sbits >= lo).astype(jnp.float32)
        mn_ref[...] = mnew
        hp = hh * (mnew * jnp.tanh(sc))
        hp_ref[...] = hp
        mx = jnp.max(jnp.where(mnew > 0, hp, -3.0e38), axis=0, keepdims=True)
        sm = jnp.sum(hp, axis=0, keepdims=True)
        xr_ref[...] = jnp.concatenate([mx, sm / kk], axis=1)

    return pl.pallas_call(
        body,
        out_shape=(jax.ShapeDtypeStruct((n, 1), jnp.float32),
                   jax.ShapeDtypeStruct((n, _D), jnp.float32),
                   jax.ShapeDtypeStruct((1, 2 * _D), jnp.float32)),
    )(h, aggp, degp, m)


def _readout(h, m, kk):
    n = h.shape[0]

    def body(h_ref, m_ref, xr_ref):
        hh = h_ref[...]
        mm = m_ref[...]
        mx = jnp.max(jnp.where(mm > 0, hh, -3.0e38), axis=0, keepdims=True)
        sm = jnp.sum(hh * mm, axis=0, keepdims=True)
        xr_ref[...] = jnp.concatenate([mx, sm / kk], axis=1)

    return pl.pallas_call(
        body,
        out_shape=jax.ShapeDtypeStruct((1, 2 * _D), jnp.float32),
    )(h, m)


def _mlp(x1, x2, x3, lw1, lb1, lw2, lb2, lw3, lb3):
    def body(x1_ref, x2_ref, x3_ref, w1_ref, c1_ref, w2_ref, c2_ref,
             w3_ref, c3_ref, out_ref):
        z = (jax.nn.relu(x1_ref[...]) + jax.nn.relu(x2_ref[...])
             + jax.nn.relu(x3_ref[...]))
        z = jax.nn.relu(jnp.dot(z, w1_ref[...],
                                preferred_element_type=jnp.float32) + c1_ref[...])
        z = jax.nn.relu(jnp.dot(z, w2_ref[...],
                                preferred_element_type=jnp.float32) + c2_ref[...])
        z = jnp.dot(z, w3_ref[...], preferred_element_type=jnp.float32) + c3_ref[...]
        out_ref[...] = jax.nn.sigmoid(z)

    return pl.pallas_call(
        body,
        out_shape=jax.ShapeDtypeStruct((1, 1), jnp.float32),
    )(x1, x2, x3, lw1, lb1, lw2, lb2, lw3, lb3)


def kernel(x, edge_index, edge_attr, pos, strata_data, batch, k,
           W1, b1, W2, b2, W3, b3, lw1, lb1, lw2, lb2, lw3, lb3):
    n = _N
    src = edge_index[0]
    dst = edge_index[1]
    ones_n = jnp.ones((n,), jnp.float32)
    k1 = n // 2
    k2 = k1 // 2

    h0 = jnp.concatenate([x, pos], axis=1)

    ew1, degp1 = _deg_pass(src, dst, edge_attr, ones_n)
    hW1, dinv1 = _mm_dinv(h0, W1, degp1)
    aggG1 = _feat_agg(hW1, src, dst, ew1, dinv1.reshape(n))
    h1 = _combine(aggG1, hW1, dinv1, b1.reshape(1, _D))
    aggI1 = _feat_agg(h1, src, dst, ew1, ones_n)
    m1, h1p, x1 = _pool(h1, aggI1, degp1, ones_n.reshape(n, 1), k1)

    ew2, degp2 = _deg_pass(src, dst, ew1, m1.reshape(n))
    hW2, dinv2 = _mm_dinv(h1p, W2, degp2)
    aggG2 = _feat_agg(hW2, src, dst, ew2, dinv2.reshape(n))
    h2 = _combine(aggG2, hW2, dinv2, b2.reshape(1, _D))
    aggI2 = _feat_agg(h2, src, dst, ew2, ones_n)
    m2, h2p, x2 = _pool(h2, aggI2, degp2, m1, k2)

    ew3, degp3 = _deg_pass(src, dst, ew2, m2.reshape(n))
    hW3, dinv3 = _mm_dinv(h2p, W3, degp3)
    aggG3 = _feat_agg(hW3, src, dst, ew3, dinv3.reshape(n))
    h3 = _combine(aggG3, hW3, dinv3, b3.reshape(1, _D))
    x3 = _readout(h3, m2, k2)

    return _mlp(x1, x2, x3, lw1, lb1.reshape(1, -1), lw2, lb2.reshape(1, -1),
                lw3, lb3.reshape(1, -1))


# speedup vs baseline: 12.0210x; 1.3187x over previous
import functools

import jax
import jax.numpy as jnp
from jax import lax
from jax.experimental import pallas as pl
from jax.experimental.pallas import tpu as pltpu
from jax.experimental.pallas import tpu_sc as plsc

_N = 10000
_E = 320000
_D = 128
_NC = 2
_NS = 16
_NW = _NC * _NS
_B = 80
_CPW = _E // (_NW * _B)
_RB = 624
_TAIL = _N - _NS * _RB
_ZR = 16
_NZ = _RB // _ZR
_L = 16


def _sc_mesh():
    return plsc.VectorSubcoreMesh(core_axis_name="c", subcore_axis_name="s")


def _zero_acc(z_v, acc_sh, sid, width):
    for r in range(_ZR):
        for c in range(width // _L):
            z_v[r, pl.ds(c * _L, _L)] = jnp.zeros((_L,), jnp.float32)

    def zloop(i, carry):
        pltpu.sync_copy(z_v, acc_sh.at[pl.ds(sid * _RB + i * _ZR, _ZR)])
        return carry

    lax.fori_loop(0, _NZ, zloop, 0)

    @pl.when(sid == 0)
    def _():
        pltpu.sync_copy(z_v, acc_sh.at[pl.ds(_NS * _RB, _TAIL)])


def _dump_acc(acc_sh, out_hbm, cid, sid):
    pltpu.sync_copy(acc_sh.at[pl.ds(sid * _RB, _RB)],
                    out_hbm.at[cid, pl.ds(sid * _RB, _RB)])

    @pl.when(sid == 0)
    def _():
        pltpu.sync_copy(acc_sh.at[pl.ds(_NS * _RB, _TAIL)],
                        out_hbm.at[cid, pl.ds(_NS * _RB, _TAIL)])


@functools.partial(
    pl.kernel,
    mesh=_sc_mesh(),
    out_type=jax.ShapeDtypeStruct((_NC, _N, _D), jnp.float32),
    scratch_types=[
        pltpu.VMEM((_E // _NW,), jnp.int32),
        pltpu.VMEM((_B,), jnp.int32),
        pltpu.VMEM((_E // _NW,), jnp.float32),
        pltpu.VMEM((2 * _L,), jnp.float32),
        pltpu.VMEM((_N,), jnp.float32),
        pltpu.VMEM((_B, _D), jnp.float32),
        pltpu.VMEM((_ZR, _D), jnp.float32),
        pltpu.VMEM_SHARED((_N, _D), jnp.float32),
        pltpu.SemaphoreType.DMA,
    ],
    compiler_params=pltpu.CompilerParams(needs_layout_passes=False),
)
def _feat_agg(h_hbm, src_hbm, dst_hbm, ew_hbm, a_hbm, out_hbm,
              src_v, dst_v, ew_v, w_v, a_v, rows_v, z_v, acc_sh, sem):
    cid = lax.axis_index("c")
    sid = lax.axis_index("s")
    wid = sid * _NC + cid

    _zero_acc(z_v, acc_sh, sid, _D)
    pltpu.sync_copy(a_hbm, a_v)
    base0 = wid * (_E // _NW)
    pltpu.sync_copy(src_hbm.at[pl.ds(base0, _E // _NW)], src_v)
    pltpu.sync_copy(ew_hbm.at[pl.ds(base0, _E // _NW)], ew_v)
    plsc.subcore_barrier()

    def chunk(i, carry):
        off = i * _B
        pltpu.sync_copy(dst_hbm.at[pl.ds(base0 + off, _B)], dst_v)
        pltpu.async_copy(h_hbm.at[src_v.at[pl.ds(off, _B)]], rows_v, sem).wait()
        for g in range(_B // _L):
            s16 = src_v[pl.ds(off + g * _L, _L)]
            e16 = ew_v[pl.ds(off + g * _L, _L)]
            w16 = e16 * plsc.load_gather(a_v, [s16])
            w_v[pl.ds(_L, _L)] = w16
            for e in range(_L):
                wb = plsc.load_gather(w_v, [jnp.full((_L,), _L + e, jnp.int32)])
                row = g * _L + e
                for c in range(_D // _L):
                    sl = pl.ds(c * _L, _L)
                    rows_v[row, sl] = rows_v[row, sl] * wb
        pltpu.sync_copy(rows_v, acc_sh.at[dst_v], add=True)
        return carry

    lax.fori_loop(0, _CPW, chunk, 0)
    plsc.subcore_barrier()
    _dump_acc(acc_sh, out_hbm, cid, sid)


@functools.partial(
    pl.kernel,
    mesh=_sc_mesh(),
    out_type=(jax.ShapeDtypeStruct((_E,), jnp.float32),
              jax.ShapeDtypeStruct((_NC, _N, _D), jnp.float32)),
    scratch_types=[
        pltpu.VMEM((_E // _NW,), jnp.int32),
        pltpu.VMEM((_B,), jnp.int32),
        pltpu.VMEM((_E // _NW,), jnp.float32),
        pltpu.VMEM((2 * _L,), jnp.float32),
        pltpu.VMEM((_N,), jnp.float32),
        pltpu.VMEM((_B, _D), jnp.float32),
        pltpu.VMEM((_ZR, _D), jnp.float32),
        pltpu.VMEM_SHARED((_N, _D), jnp.float32),
    ],
    compiler_params=pltpu.CompilerParams(needs_layout_passes=False),
)
def _deg_pass(src_hbm, dst_hbm, ew_hbm, m_hbm, ewout_hbm, deg_hbm,
              src_v, dst_v, ew_v, stg_v, m_v, rows_v, z_v, acc_sh):
    cid = lax.axis_index("c")
    sid = lax.axis_index("s")
    wid = sid * _NC + cid

    _zero_acc(z_v, acc_sh, sid, _D)
    pltpu.sync_copy(m_hbm, m_v)
    base0 = wid * (_E // _NW)
    pltpu.sync_copy(src_hbm.at[pl.ds(base0, _E // _NW)], src_v)
    pltpu.sync_copy(ew_hbm.at[pl.ds(base0, _E // _NW)], ew_v)
    plsc.subcore_barrier()

    def chunk(i, carry):
        off = i * _B
        pltpu.sync_copy(dst_hbm.at[pl.ds(base0 + off, _B)], dst_v)
        for g in range(_B // _L):
            s16 = src_v[pl.ds(off + g * _L, _L)]
            d16 = dst_v[pl.ds(g * _L, _L)]
            e16 = ew_v[pl.ds(off + g * _L, _L)]
            w16 = e16 * plsc.load_gather(m_v, [s16]) * plsc.load_gather(m_v, [d16])
            ew_v[pl.ds(off + g * _L, _L)] = w16
            stg_v[pl.ds(_L, _L)] = w16
            for e in range(_L):
                row = g * _L + e
                wb = plsc.load_gather(stg_v, [jnp.full((_L,), _L + e, jnp.int32)])
                for c in range(_D // _L):
                    rows_v[row, pl.ds(c * _L, _L)] = wb
        pltpu.sync_copy(rows_v, acc_sh.at[dst_v], add=True)
        return carry

    lax.fori_loop(0, _CPW, chunk, 0)
    pltpu.sync_copy(ew_v, ewout_hbm.at[pl.ds(base0, _E // _NW)])
    plsc.subcore_barrier()
    _dump_acc(acc_sh, deg_hbm, cid, sid)


def _mm_dinv(h, W, degp):
    n = h.shape[0]

    def body(h_ref, w_ref, degp_ref, hw_ref, dinv_ref):
        hw_ref[...] = jnp.dot(h_ref[...], w_ref[...],
                              preferred_element_type=jnp.float32)
        degn = degp_ref[0, :, 0:1] + degp_ref[1, :, 0:1]
        dinv_ref[...] = lax.rsqrt(degn + 1.0)

    return pl.pallas_call(
        body,
        out_shape=(jax.ShapeDtypeStruct((n, _D), jnp.float32),
                   jax.ShapeDtypeStruct((n, 1), jnp.float32)),
    )(h, W, degp)


def _combine(aggp, hw, dinv, b):
    n = hw.shape[0]

    def body(aggp_ref, hw_ref, dinv_ref, b_ref, out_ref):
        agg = aggp_ref[0] + aggp_ref[1]
        dv = dinv_ref[...]
        out_ref[...] = jax.nn.relu(dv * agg + dv * dv * hw_ref[...] + b_ref[...])

    return pl.pallas_call(
        body,
        out_shape=jax.ShapeDtypeStruct((n, _D), jnp.float32),
    )(aggp, hw, dinv, b)


def _pool(h, aggp, degp, m, kk):
    n = h.shape[0]

    def body(h_ref, aggp_ref, degp_ref, m_ref, mn_ref, hp_ref, xr_ref):
        degn = degp_ref[0, :, 0:1] + degp_ref[1, :, 0:1]
        degs = jnp.where(degn > 0, degn, 1.0)
        neigh = (aggp_ref[0] + aggp_ref[1]) / degs
        hh = h_ref[...]
        sc = jnp.sum(jnp.abs(hh - neigh), axis=1, keepdims=True)
        sbits = jnp.where(m_ref[...] > 0,
                          lax.bitcast_convert_type(sc, jnp.int32),
                          jnp.int32(-1))

        def bis(i, lohi):
            lo, hi = lohi
            mid = lo + (hi - lo + 1) // 2
            cnt = jnp.sum((sbits >= mid).astype(jnp.int32))
            ok = cnt >= kk
            return (jnp.where(ok, mid, lo), jnp.where(ok, hi, mid - 1))

        lo, _ = lax.fori_loop(0, 31, bis,
                              (jnp.int32(0), jnp.int32(0x7F800000)))
        mnew = (sbits >= lo).astype(jnp.float32)
        mn_ref[...] = mnew
        hp = hh * (mnew * jnp.tanh(sc))
        hp_ref[...] = hp
        mx = jnp.max(jnp.where(mnew > 0, hp, -3.0e38), axis=0, keepdims=True)
        sm = jnp.sum(hp, axis=0, keepdims=True)
        xr_ref[...] = jnp.concatenate([mx, sm / kk], axis=1)

    return pl.pallas_call(
        body,
        out_shape=(jax.ShapeDtypeStruct((n, 1), jnp.float32),
                   jax.ShapeDtypeStruct((n, _D), jnp.float32),
                   jax.ShapeDtypeStruct((1, 2 * _D), jnp.float32)),
    )(h, aggp, degp, m)


def _readout(h, m, kk):
    n = h.shape[0]

    def body(h_ref, m_ref, xr_ref):
        hh = h_ref[...]
        mm = m_ref[...]
        mx = jnp.max(jnp.where(mm > 0, hh, -3.0e38), axis=0, keepdims=True)
        sm = jnp.sum(hh * mm, axis=0, keepdims=True)
        xr_ref[...] = jnp.concatenate([mx, sm / kk], axis=1)

    return pl.pallas_call(
        body,
        out_shape=jax.ShapeDtypeStruct((1, 2 * _D), jnp.float32),
    )(h, m)


def _mlp(x1, x2, x3, lw1, lb1, lw2, lb2, lw3, lb3):
    def body(x1_ref, x2_ref, x3_ref, w1_ref, c1_ref, w2_ref, c2_ref,
             w3_ref, c3_ref, out_ref):
        z = (jax.nn.relu(x1_ref[...]) + jax.nn.relu(x2_ref[...])
             + jax.nn.relu(x3_ref[...]))
        z = jax.nn.relu(jnp.dot(z, w1_ref[...],
                                preferred_element_type=jnp.float32) + c1_ref[...])
        z = jax.nn.relu(jnp.dot(z, w2_ref[...],
                                preferred_element_type=jnp.float32) + c2_ref[...])
        z = jnp.dot(z, w3_ref[...], preferred_element_type=jnp.float32) + c3_ref[...]
        out_ref[...] = jax.nn.sigmoid(z)

    return pl.pallas_call(
        body,
        out_shape=jax.ShapeDtypeStruct((1, 1), jnp.float32),
    )(x1, x2, x3, lw1, lb1, lw2, lb2, lw3, lb3)


def kernel(x, edge_index, edge_attr, pos, strata_data, batch, k,
           W1, b1, W2, b2, W3, b3, lw1, lb1, lw2, lb2, lw3, lb3):
    n = _N
    src = edge_index[0]
    dst = edge_index[1]
    ones_n = jnp.ones((n,), jnp.float32)
    k1 = n // 2
    k2 = k1 // 2

    h0 = jnp.concatenate([x, pos], axis=1)

    ew1, degp1 = _deg_pass(src, dst, edge_attr, ones_n)
    hW1, dinv1 = _mm_dinv(h0, W1, degp1)
    aggG1 = _feat_agg(hW1, src, dst, ew1, dinv1.reshape(n))
    h1 = _combine(aggG1, hW1, dinv1, b1.reshape(1, _D))
    aggI1 = _feat_agg(h1, src, dst, ew1, ones_n)
    m1, h1p, x1 = _pool(h1, aggI1, degp1, ones_n.reshape(n, 1), k1)

    ew2, degp2 = _deg_pass(src, dst, ew1, m1.reshape(n))
    hW2, dinv2 = _mm_dinv(h1p, W2, degp2)
    aggG2 = _feat_agg(hW2, src, dst, ew2, dinv2.reshape(n))
    h2 = _combine(aggG2, hW2, dinv2, b2.reshape(1, _D))
    aggI2 = _feat_agg(h2, src, dst, ew2, ones_n)
    m2, h2p, x2 = _pool(h2, aggI2, degp2, m1, k2)

    ew3, degp3 = _deg_pass(src, dst, ew2, m2.reshape(n))
    hW3, dinv3 = _mm_dinv(h2p, W3, degp3)
    aggG3 = _feat_agg(hW3, src, dst, ew3, dinv3.reshape(n))
    h3 = _combine(aggG3, hW3, dinv3, b3.reshape(1, _D))
    x3 = _readout(h3, m2, k2)

    return _mlp(x1, x2, x3, lw1, lb1.reshape(1, -1), lw2, lb2.reshape(1, -1),
                lw3, lb3.reshape(1, -1))
